# baseline (speedup 1.0000x reference)
import functools
import math

import numpy as np
import jax
import jax.numpy as jnp
from jax.experimental import pallas as pl
from jax.experimental.pallas import tpu as pltpu

_SR = 22050
_F_MIN = 32.7
_BPO = 12
_HOP = 256


@functools.lru_cache(maxsize=None)
def _cqt_constants():
    f_max = _SR / 2.0
    q = 1.0 / (2.0 ** (1.0 / _BPO) - 1.0)
    n_bins = math.ceil(_BPO * math.log2(f_max / _F_MIN))
    fft_len = 1 << (int(math.ceil(q * _SR / _F_MIN)) - 1).bit_length()

    temporal = np.zeros((n_bins, fft_len), dtype=np.complex128)
    for k in range(n_bins):
        f_k = _F_MIN * 2.0 ** (k / _BPO)
        n_k = 2 * round(q * _SR / f_k / 2) + 1
        n = np.arange(-(n_k - 1) // 2, (n_k - 1) // 2 + 1)
        w = np.hamming(n_k) / n_k
        start = fft_len // 2 + n[0]
        temporal[k, start:start + n_k] = w * np.exp(2j * np.pi * q / n_k * n)
    spectral = np.fft.fft(temporal, axis=-1) / fft_len
    folded = np.fft.fft(spectral, axis=-1).T

    c_int = np.zeros((fft_len, 2 * n_bins), dtype=np.float64)
    c_int[:, 0::2] = folded.real
    c_int[:, 1::2] = folded.imag

    row_amp = np.abs(c_int).max(axis=1)
    nz = np.nonzero(row_amp > row_amp.max() * 1e-7)[0]
    s0 = int(nz[0]) // _HOP
    s1 = int(nz[-1]) // _HOP + 1
    ns = s1 - s0

    c_pad = np.zeros((fft_len, 256), np.float64)
    c_pad[:, :2 * n_bins] = c_int
    hi_amp = np.abs(c_pad[:, 128:]).max(axis=1)
    hz = np.nonzero(hi_amp > row_amp.max() * 1e-7)[0]
    hr0 = int(hz[0]) // _HOP - s0
    hr1 = int(hz[-1]) // _HOP + 1 - s0

    def _band(cols):
        amp = np.abs(c_pad[:, cols]).max(axis=1)
        nzc = np.nonzero(amp > row_amp.max() * 1e-7)[0]
        return int(nzc[0]) // _HOP - s0, int(nzc[-1]) // _HOP + 1 - s0

    aS, aE = _band(slice(0, 64))
    bS, bE = _band(slice(64, 128))
    k1 = max(-(-(aE - aS) // 3), bE - bS)
    nv = k1
    c_d1 = np.zeros((k1 * _HOP, 256), np.float32)
    for t in range(k1):
        for m in range(3):
            s = aS + m * k1 + t
            if s < ns:
                c_d1[t * _HOP:(t + 1) * _HOP, 64 * m:64 * (m + 1)] = (
                    c_pad[(s0 + s) * _HOP:(s0 + s + 1) * _HOP, :64])
        s = bS + t
        if s < bE:
            c_d1[t * _HOP:(t + 1) * _HOP, 192:256] = (
                c_pad[(s0 + s) * _HOP:(s0 + s + 1) * _HOP, 64:128])
    nh = hr1 - hr0
    c_d2 = np.zeros((nh * _HOP, 128), np.float32)
    for t in range(nh):
        c_d2[t * _HOP:(t + 1) * _HOP] = c_pad[(s0 + hr0 + t) * _HOP:
                                              (s0 + hr0 + t + 1) * _HOP, 128:]
    return {
        "n_bins": n_bins,
        "fft_len": fft_len,
        "s0": s0,
        "ns": ns,
        "k1": k1,
        "aS": aS,
        "bS": bS,
        "hr0": hr0,
        "nh": nh,
        "nv": nv,
        "c_d1": jnp.asarray(c_d1, jnp.bfloat16),
        "c_d2": jnp.asarray(c_d2, jnp.bfloat16),
    }


def _cqt_pallas(y, c_d1, c_d2, *, n_frames, cst, kout):
    batch = y.shape[0]
    yrows = y.shape[1]
    hop = y.shape[2]
    k1, aS, bS = cst["k1"], cst["aS"], cst["bS"]
    hr0, nh, nv = cst["hr0"], cst["nh"], cst["nv"]
    off_max = max(aS + 2 * k1, bS, hr0)
    mp = -(-(n_frames + off_max) // 8) * 8
    zrows = -(-mp // 16) * 16

    def body(y_ref, c1_ref, c2_ref, o_ref, zbuf):
        for v in range(nv):
            zbuf[:, v * hop:(v + 1) * hop] = y_ref[0, v:v + zrows, :]
        d1 = jnp.dot(zbuf[0:mp, :], c1_ref[...],
                     preferred_element_type=jnp.float32)
        d2 = jnp.dot(zbuf[0:mp, 0:nh * hop], c2_ref[...],
                     preferred_element_type=jnp.float32)
        lo_a = (d1[aS:aS + n_frames, 0:64]
                + d1[aS + k1:aS + k1 + n_frames, 64:128]
                + d1[aS + 2 * k1:aS + 2 * k1 + n_frames, 128:192])
        lo_b = d1[bS:bS + n_frames, 192:256]
        hi = d2[hr0:hr0 + n_frames, 0:kout - 128]
        o_ref[0] = jnp.concatenate([lo_a, lo_b, hi], axis=1)

    return pl.pallas_call(
        body,
        out_shape=jax.ShapeDtypeStruct((batch, n_frames, kout), jnp.float32),
        grid=(batch,),
        in_specs=[
            pl.BlockSpec((1, yrows, hop), lambda b: (b, 0, 0)),
            pl.BlockSpec(c_d1.shape, lambda b: (0, 0)),
            pl.BlockSpec(c_d2.shape, lambda b: (0, 0)),
        ],
        out_specs=pl.BlockSpec((1, n_frames, kout), lambda b: (b, 0, 0)),
        scratch_shapes=[
            pltpu.VMEM((zrows, nv * hop), jnp.bfloat16),
        ],
        compiler_params=pltpu.CompilerParams(
            dimension_semantics=("parallel",)),
    )(y, c_d1, c_d2)


def kernel(x):
    cst = _cqt_constants()
    n_bins, fft_len = cst["n_bins"], cst["fft_len"]
    s0 = cst["s0"]

    x = jnp.asarray(x, jnp.float32)
    lead, t_len = x.shape[:-1], x.shape[-1]
    x2 = x.reshape(-1, t_len)
    batch = x2.shape[0]
    n_frames = (t_len - 1) // _HOP + 1

    off_max = max(cst["aS"] + 2 * cst["k1"], cst["bS"], cst["hr0"])
    mp = -(-(n_frames + off_max) // 8) * 8
    zrows = -(-mp // 16) * 16
    yrows = -(-(zrows + cst["nv"] - 1) // 16) * 16
    lpad = fft_len // 2 - s0 * _HOP
    y = jnp.pad(x2.astype(jnp.bfloat16),
                ((0, 0), (lpad, yrows * _HOP - lpad - t_len)))
    y = y.reshape(batch, yrows, _HOP)

    out = _cqt_pallas(y, cst["c_d1"], cst["c_d2"], n_frames=n_frames,
                      cst=cst, kout=2 * n_bins)
    return out.reshape(*lead, n_frames, n_bins, 2)


# speedup vs baseline: 1.0423x; 1.0423x over previous
import functools
import math

import numpy as np
import jax
import jax.numpy as jnp
from jax.experimental import pallas as pl
from jax.experimental.pallas import tpu as pltpu

_SR = 22050
_F_MIN = 32.7
_BPO = 12
_HOP = 256


@functools.lru_cache(maxsize=None)
def _cqt_constants():
    f_max = _SR / 2.0
    q = 1.0 / (2.0 ** (1.0 / _BPO) - 1.0)
    n_bins = math.ceil(_BPO * math.log2(f_max / _F_MIN))
    fft_len = 1 << (int(math.ceil(q * _SR / _F_MIN)) - 1).bit_length()

    temporal = np.zeros((n_bins, fft_len), dtype=np.complex128)
    for k in range(n_bins):
        f_k = _F_MIN * 2.0 ** (k / _BPO)
        n_k = 2 * round(q * _SR / f_k / 2) + 1
        n = np.arange(-(n_k - 1) // 2, (n_k - 1) // 2 + 1)
        w = np.hamming(n_k) / n_k
        start = fft_len // 2 + n[0]
        temporal[k, start:start + n_k] = w * np.exp(2j * np.pi * q / n_k * n)
    spectral = np.fft.fft(temporal, axis=-1) / fft_len
    folded = np.fft.fft(spectral, axis=-1).T

    c_int = np.zeros((fft_len, 2 * n_bins), dtype=np.float64)
    c_int[:, 0::2] = folded.real
    c_int[:, 1::2] = folded.imag

    row_amp = np.abs(c_int).max(axis=1)
    nz = np.nonzero(row_amp > row_amp.max() * 1e-7)[0]
    s0 = int(nz[0]) // _HOP
    s1 = int(nz[-1]) // _HOP + 1
    ns = s1 - s0

    c_pad = np.zeros((fft_len, 256), np.float64)
    c_pad[:, :2 * n_bins] = c_int
    hi_amp = np.abs(c_pad[:, 128:]).max(axis=1)
    hz = np.nonzero(hi_amp > row_amp.max() * 1e-7)[0]
    hr0 = int(hz[0]) // _HOP - s0
    hr1 = int(hz[-1]) // _HOP + 1 - s0

    def _band(cols):
        amp = np.abs(c_pad[:, cols]).max(axis=1)
        nzc = np.nonzero(amp > row_amp.max() * 1e-7)[0]
        return int(nzc[0]) // _HOP - s0, int(nzc[-1]) // _HOP + 1 - s0

    aS, aE = _band(slice(0, 32))
    cS, cE = _band(slice(32, 64))
    bS, bE = _band(slice(64, 128))
    k1 = max(-(-(aE - aS) // 4), -(-(cE - cS) // 2), bE - bS)
    nv = k1
    c_d1 = np.zeros((k1 * _HOP, 256), np.float32)
    for t in range(k1):
        for m in range(4):
            s = aS + m * k1 + t
            if s < min(aE, ns):
                c_d1[t * _HOP:(t + 1) * _HOP, 32 * m:32 * (m + 1)] = (
                    c_pad[(s0 + s) * _HOP:(s0 + s + 1) * _HOP, :32])
        for m in range(2):
            s = cS + m * k1 + t
            if s < min(cE, ns):
                c_d1[t * _HOP:(t + 1) * _HOP, 128 + 32 * m:160 + 32 * m] = (
                    c_pad[(s0 + s) * _HOP:(s0 + s + 1) * _HOP, 32:64])
        s = bS + t
        if s < bE:
            c_d1[t * _HOP:(t + 1) * _HOP, 192:256] = (
                c_pad[(s0 + s) * _HOP:(s0 + s + 1) * _HOP, 64:128])
    nh = hr1 - hr0
    c_d2 = np.zeros((nh * _HOP, 128), np.float32)
    for t in range(nh):
        c_d2[t * _HOP:(t + 1) * _HOP] = c_pad[(s0 + hr0 + t) * _HOP:
                                              (s0 + hr0 + t + 1) * _HOP, 128:]
    return {
        "n_bins": n_bins,
        "fft_len": fft_len,
        "s0": s0,
        "ns": ns,
        "k1": k1,
        "aS": aS,
        "cS": cS,
        "bS": bS,
        "hr0": hr0,
        "nh": nh,
        "nv": nv,
        "c_d1": jnp.asarray(c_d1, jnp.bfloat16),
        "c_d2": jnp.asarray(c_d2, jnp.bfloat16),
    }


def _cqt_pallas(x4, c_d1, c_d2, *, n_frames, lead_zero, sig_rows, cst, kout):
    batch = x4.shape[0]
    hop = x4.shape[2]
    k1, aS, cS, bS = cst["k1"], cst["aS"], cst["cS"], cst["bS"]
    hr0, nh, nv = cst["hr0"], cst["nh"], cst["nv"]
    off_max = max(aS + 3 * k1, cS + k1, bS, hr0)
    mp = -(-(n_frames + off_max) // 8) * 8
    zrows = -(-mp // 16) * 16
    yrows = -(-(zrows + nv - 1) // 16) * 16

    def body(x_ref, c1_ref, c2_ref, o_ref, ybuf, zbuf):
        ybuf[:lead_zero, :] = jnp.zeros((lead_zero, hop), jnp.bfloat16)
        ybuf[lead_zero:lead_zero + sig_rows, :] = x_ref[0].astype(jnp.bfloat16)
        ybuf[lead_zero + sig_rows:, :] = jnp.zeros(
            (yrows - lead_zero - sig_rows, hop), jnp.bfloat16)
        for v in range(nv):
            zbuf[:, v * hop:(v + 1) * hop] = ybuf[v:v + zrows, :]
        d1 = jnp.dot(zbuf[0:mp, :], c1_ref[...],
                     preferred_element_type=jnp.float32)
        d2 = jnp.dot(zbuf[0:mp, 0:nh * hop], c2_ref[...],
                     preferred_element_type=jnp.float32)
        lo_a = (d1[aS:aS + n_frames, 0:32]
                + d1[aS + k1:aS + k1 + n_frames, 32:64]
                + d1[aS + 2 * k1:aS + 2 * k1 + n_frames, 64:96]
                + d1[aS + 3 * k1:aS + 3 * k1 + n_frames, 96:128])
        lo_c = (d1[cS:cS + n_frames, 128:160]
                + d1[cS + k1:cS + k1 + n_frames, 160:192])
        lo_b = d1[bS:bS + n_frames, 192:256]
        hi = d2[hr0:hr0 + n_frames, 0:kout - 128]
        o_ref[0] = jnp.concatenate([lo_a, lo_c, lo_b, hi], axis=1)

    return pl.pallas_call(
        body,
        out_shape=jax.ShapeDtypeStruct((batch, n_frames, kout), jnp.float32),
        grid=(batch,),
        in_specs=[
            pl.BlockSpec((1, sig_rows, hop), lambda b: (b, 0, 0)),
            pl.BlockSpec(c_d1.shape, lambda b: (0, 0)),
            pl.BlockSpec(c_d2.shape, lambda b: (0, 0)),
        ],
        out_specs=pl.BlockSpec((1, n_frames, kout), lambda b: (b, 0, 0)),
        scratch_shapes=[
            pltpu.VMEM((yrows, hop), jnp.bfloat16),
            pltpu.VMEM((zrows, nv * hop), jnp.bfloat16),
        ],
        compiler_params=pltpu.CompilerParams(
            dimension_semantics=("parallel",)),
    )(x4, c_d1, c_d2)


def kernel(x):
    cst = _cqt_constants()
    n_bins, fft_len = cst["n_bins"], cst["fft_len"]
    s0 = cst["s0"]

    x = jnp.asarray(x, jnp.float32)
    lead, t_len = x.shape[:-1], x.shape[-1]
    x2 = x.reshape(-1, t_len)
    batch = x2.shape[0]
    n_frames = (t_len - 1) // _HOP + 1

    sig_rows = -(-t_len // _HOP)
    if t_len % _HOP:
        x2 = jnp.pad(x2, ((0, 0), (0, sig_rows * _HOP - t_len)))
    x4 = x2.reshape(batch, sig_rows, _HOP)

    lead_zero = fft_len // 2 // _HOP - s0
    out = _cqt_pallas(x4, cst["c_d1"], cst["c_d2"], n_frames=n_frames,
                      lead_zero=lead_zero, sig_rows=sig_rows, cst=cst,
                      kout=2 * n_bins)
    return out.reshape(*lead, n_frames, n_bins, 2)


# speedup vs baseline: 1.1607x; 1.1136x over previous
import functools
import math

import numpy as np
import jax
import jax.numpy as jnp
from jax.experimental import pallas as pl
from jax.experimental.pallas import tpu as pltpu

_SR = 22050
_F_MIN = 32.7
_BPO = 12
_HOP = 256


@functools.lru_cache(maxsize=None)
def _cqt_constants():
    f_max = _SR / 2.0
    q = 1.0 / (2.0 ** (1.0 / _BPO) - 1.0)
    n_bins = math.ceil(_BPO * math.log2(f_max / _F_MIN))
    fft_len = 1 << (int(math.ceil(q * _SR / _F_MIN)) - 1).bit_length()

    temporal = np.zeros((n_bins, fft_len), dtype=np.complex128)
    for k in range(n_bins):
        f_k = _F_MIN * 2.0 ** (k / _BPO)
        n_k = 2 * round(q * _SR / f_k / 2) + 1
        n = np.arange(-(n_k - 1) // 2, (n_k - 1) // 2 + 1)
        w = np.hamming(n_k) / n_k
        start = fft_len // 2 + n[0]
        temporal[k, start:start + n_k] = w * np.exp(2j * np.pi * q / n_k * n)
    spectral = np.fft.fft(temporal, axis=-1) / fft_len
    folded = np.fft.fft(spectral, axis=-1).T

    c_int = np.zeros((fft_len, 2 * n_bins), dtype=np.float64)
    c_int[:, 0::2] = folded.real
    c_int[:, 1::2] = folded.imag

    row_amp = np.abs(c_int).max(axis=1)
    nz = np.nonzero(row_amp > row_amp.max() * 1e-7)[0]
    s0 = int(nz[0]) // _HOP
    s1 = int(nz[-1]) // _HOP + 1
    ns = s1 - s0

    c_pad = np.zeros((fft_len, 256), np.float64)
    c_pad[:, :2 * n_bins] = c_int
    hi_amp = np.abs(c_pad[:, 128:]).max(axis=1)
    hz = np.nonzero(hi_amp > row_amp.max() * 1e-7)[0]
    hr0 = int(hz[0]) // _HOP - s0
    hr1 = int(hz[-1]) // _HOP + 1 - s0

    def _band(cols):
        amp = np.abs(c_pad[:, cols]).max(axis=1)
        nzc = np.nonzero(amp > row_amp.max() * 1e-7)[0]
        return int(nzc[0]) // _HOP - s0, int(nzc[-1]) // _HOP + 1 - s0

    aS, aE = _band(slice(0, 32))
    cS, cE = _band(slice(32, 64))
    bS, bE = _band(slice(64, 128))
    k1 = max(-(-(aE - aS) // 4), -(-(cE - cS) // 2), bE - bS)
    nv = k1
    c_d1 = np.zeros((k1 * _HOP, 256), np.float32)
    for t in range(k1):
        for m in range(4):
            s = aS + m * k1 + t
            if s < min(aE, ns):
                c_d1[t * _HOP:(t + 1) * _HOP, 32 * m:32 * (m + 1)] = (
                    c_pad[(s0 + s) * _HOP:(s0 + s + 1) * _HOP, :32])
        for m in range(2):
            s = cS + m * k1 + t
            if s < min(cE, ns):
                c_d1[t * _HOP:(t + 1) * _HOP, 128 + 32 * m:160 + 32 * m] = (
                    c_pad[(s0 + s) * _HOP:(s0 + s + 1) * _HOP, 32:64])
        s = bS + t
        if s < bE:
            c_d1[t * _HOP:(t + 1) * _HOP, 192:256] = (
                c_pad[(s0 + s) * _HOP:(s0 + s + 1) * _HOP, 64:128])
    nh = hr1 - hr0
    c_d2 = np.zeros((nh * _HOP, 128), np.float32)
    for t in range(nh):
        c_d2[t * _HOP:(t + 1) * _HOP] = c_pad[(s0 + hr0 + t) * _HOP:
                                              (s0 + hr0 + t + 1) * _HOP, 128:]
    return {
        "n_bins": n_bins,
        "fft_len": fft_len,
        "s0": s0,
        "ns": ns,
        "k1": k1,
        "aS": aS,
        "cS": cS,
        "bS": bS,
        "hr0": hr0,
        "nh": nh,
        "nv": nv,
        "c_d1": jnp.asarray(c_d1, jnp.bfloat16),
        "c_d2": jnp.asarray(c_d2, jnp.bfloat16),
    }


def _cqt_pallas(x4, c_d1, c_d2, *, n_frames, lead_zero, sig_rows, cst, kout):
    batch = x4.shape[0]
    hop = x4.shape[2]
    k1, aS, cS, bS = cst["k1"], cst["aS"], cst["cS"], cst["bS"]
    hr0, nh, nv = cst["hr0"], cst["nh"], cst["nv"]
    off_max = max(aS + 3 * k1, cS + k1, bS, hr0)
    mp = -(-(n_frames + off_max) // 8) * 8
    zrows = -(-mp // 16) * 16
    yrows = -(-(zrows + nv - 1) // 16) * 16

    bpg = 2 if batch % 2 == 0 else 1

    def body(x_ref, c1_ref, c2_ref, o_ref, ybuf, zbuf):
        for bb in range(bpg):
            ybuf[:lead_zero, :] = jnp.zeros((lead_zero, hop), jnp.bfloat16)
            ybuf[lead_zero:lead_zero + sig_rows, :] = (
                x_ref[bb].astype(jnp.bfloat16))
            ybuf[lead_zero + sig_rows:, :] = jnp.zeros(
                (yrows - lead_zero - sig_rows, hop), jnp.bfloat16)
            for v in range(nv):
                zbuf[bb][:, v * hop:(v + 1) * hop] = ybuf[v:v + zrows, :]
        for bb in range(bpg):
            d1 = jnp.dot(zbuf[bb][0:mp, :], c1_ref[...],
                         preferred_element_type=jnp.float32)
            d2 = jnp.dot(zbuf[bb][0:mp, 0:nh * hop], c2_ref[...],
                         preferred_element_type=jnp.float32)
            lo_a = (d1[aS:aS + n_frames, 0:32]
                    + d1[aS + k1:aS + k1 + n_frames, 32:64]
                    + d1[aS + 2 * k1:aS + 2 * k1 + n_frames, 64:96]
                    + d1[aS + 3 * k1:aS + 3 * k1 + n_frames, 96:128])
            lo_c = (d1[cS:cS + n_frames, 128:160]
                    + d1[cS + k1:cS + k1 + n_frames, 160:192])
            lo_b = d1[bS:bS + n_frames, 192:256]
            hi = d2[hr0:hr0 + n_frames, 0:kout - 128]
            o_ref[bb] = jnp.concatenate([lo_a, lo_c, lo_b, hi], axis=1)

    return pl.pallas_call(
        body,
        out_shape=jax.ShapeDtypeStruct((batch, n_frames, kout), jnp.float32),
        grid=(batch // bpg,),
        in_specs=[
            pl.BlockSpec((bpg, sig_rows, hop), lambda b: (b, 0, 0)),
            pl.BlockSpec(c_d1.shape, lambda b: (0, 0)),
            pl.BlockSpec(c_d2.shape, lambda b: (0, 0)),
        ],
        out_specs=pl.BlockSpec((bpg, n_frames, kout), lambda b: (b, 0, 0)),
        scratch_shapes=[
            pltpu.VMEM((yrows, hop), jnp.bfloat16),
            [pltpu.VMEM((zrows, nv * hop), jnp.bfloat16)
             for _ in range(bpg)],
        ],
        compiler_params=pltpu.CompilerParams(
            dimension_semantics=("parallel",)),
    )(x4, c_d1, c_d2)


def kernel(x):
    cst = _cqt_constants()
    n_bins, fft_len = cst["n_bins"], cst["fft_len"]
    s0 = cst["s0"]

    x = jnp.asarray(x, jnp.float32)
    lead, t_len = x.shape[:-1], x.shape[-1]
    x2 = x.reshape(-1, t_len)
    batch = x2.shape[0]
    n_frames = (t_len - 1) // _HOP + 1

    sig_rows = -(-t_len // _HOP)
    if t_len % _HOP:
        x2 = jnp.pad(x2, ((0, 0), (0, sig_rows * _HOP - t_len)))
    x4 = x2.reshape(batch, sig_rows, _HOP)

    lead_zero = fft_len // 2 // _HOP - s0
    out = _cqt_pallas(x4, cst["c_d1"], cst["c_d2"], n_frames=n_frames,
                      lead_zero=lead_zero, sig_rows=sig_rows, cst=cst,
                      kout=2 * n_bins)
    return out.reshape(*lead, n_frames, n_bins, 2)


# speedup vs baseline: 1.2158x; 1.0474x over previous
import functools
import math

import numpy as np
import jax
import jax.numpy as jnp
from jax.experimental import pallas as pl
from jax.experimental.pallas import tpu as pltpu

_SR = 22050
_F_MIN = 32.7
_BPO = 12
_HOP = 256


@functools.lru_cache(maxsize=None)
def _cqt_constants():
    f_max = _SR / 2.0
    q = 1.0 / (2.0 ** (1.0 / _BPO) - 1.0)
    n_bins = math.ceil(_BPO * math.log2(f_max / _F_MIN))
    fft_len = 1 << (int(math.ceil(q * _SR / _F_MIN)) - 1).bit_length()

    temporal = np.zeros((n_bins, fft_len), dtype=np.complex128)
    for k in range(n_bins):
        f_k = _F_MIN * 2.0 ** (k / _BPO)
        n_k = 2 * round(q * _SR / f_k / 2) + 1
        n = np.arange(-(n_k - 1) // 2, (n_k - 1) // 2 + 1)
        w = np.hamming(n_k) / n_k
        start = fft_len // 2 + n[0]
        temporal[k, start:start + n_k] = w * np.exp(2j * np.pi * q / n_k * n)
    spectral = np.fft.fft(temporal, axis=-1) / fft_len
    folded = np.fft.fft(spectral, axis=-1).T

    c_int = np.zeros((fft_len, 2 * n_bins), dtype=np.float64)
    c_int[:, 0::2] = folded.real
    c_int[:, 1::2] = folded.imag

    row_amp = np.abs(c_int).max(axis=1)
    nz = np.nonzero(row_amp > row_amp.max() * 1e-7)[0]
    s0 = int(nz[0]) // _HOP
    s1 = int(nz[-1]) // _HOP + 1
    ns = s1 - s0

    c_pad = np.zeros((fft_len, 256), np.float64)
    c_pad[:, :2 * n_bins] = c_int
    hi_amp = np.abs(c_pad[:, 128:]).max(axis=1)
    hz = np.nonzero(hi_amp > row_amp.max() * 1e-7)[0]
    hr0 = int(hz[0]) // _HOP - s0
    hr1 = int(hz[-1]) // _HOP + 1 - s0

    def _band(cols):
        amp = np.abs(c_pad[:, cols]).max(axis=1)
        nzc = np.nonzero(amp > row_amp.max() * 1e-7)[0]
        return int(nzc[0]) // _HOP - s0, int(nzc[-1]) // _HOP + 1 - s0

    aS, aE = _band(slice(0, 32))
    cS, cE = _band(slice(32, 64))
    bS, bE = _band(slice(64, 128))
    k1 = max(-(-(aE - aS) // 4), -(-(cE - cS) // 2), bE - bS)
    nv = k1
    c_d1 = np.zeros((k1 * _HOP, 256), np.float32)
    for t in range(k1):
        for m in range(4):
            s = aS + m * k1 + t
            if s < min(aE, ns):
                c_d1[t * _HOP:(t + 1) * _HOP, 32 * m:32 * (m + 1)] = (
                    c_pad[(s0 + s) * _HOP:(s0 + s + 1) * _HOP, :32])
        for m in range(2):
            s = cS + m * k1 + t
            if s < min(cE, ns):
                c_d1[t * _HOP:(t + 1) * _HOP, 128 + 32 * m:160 + 32 * m] = (
                    c_pad[(s0 + s) * _HOP:(s0 + s + 1) * _HOP, 32:64])
        s = bS + t
        if s < bE:
            c_d1[t * _HOP:(t + 1) * _HOP, 192:256] = (
                c_pad[(s0 + s) * _HOP:(s0 + s + 1) * _HOP, 64:128])
    nh = hr1 - hr0
    c_d2 = np.zeros((nh * _HOP, 128), np.float32)
    for t in range(nh):
        c_d2[t * _HOP:(t + 1) * _HOP] = c_pad[(s0 + hr0 + t) * _HOP:
                                              (s0 + hr0 + t + 1) * _HOP, 128:]
    return {
        "n_bins": n_bins,
        "fft_len": fft_len,
        "s0": s0,
        "ns": ns,
        "k1": k1,
        "aS": aS,
        "cS": cS,
        "bS": bS,
        "hr0": hr0,
        "nh": nh,
        "nv": nv,
        "c_d1": jnp.asarray(c_d1, jnp.bfloat16),
        "c_d2": jnp.asarray(c_d2, jnp.bfloat16),
    }


def _cqt_pallas(x4, c_d1, c_d2, *, n_frames, lead_zero, sig_rows, cst, kout):
    batch = x4.shape[0]
    hop = x4.shape[2]
    k1, aS, cS, bS = cst["k1"], cst["aS"], cst["cS"], cst["bS"]
    hr0, nh, nv = cst["hr0"], cst["nh"], cst["nv"]
    off_max = max(aS + 3 * k1, cS + k1, bS, hr0)
    mp = -(-(n_frames + off_max) // 8) * 8
    zrows = -(-mp // 16) * 16
    yrows = -(-(zrows + nv - 1) // 16) * 16

    bpg = 4 if batch % 4 == 0 else (2 if batch % 2 == 0 else 1)

    def body(x_ref, c1_ref, c2_ref, o_ref, ybuf, zbuf):
        for bb in range(bpg):
            ybuf[:lead_zero, :] = jnp.zeros((lead_zero, hop), jnp.bfloat16)
            ybuf[lead_zero:lead_zero + sig_rows, :] = (
                x_ref[bb].astype(jnp.bfloat16))
            ybuf[lead_zero + sig_rows:, :] = jnp.zeros(
                (yrows - lead_zero - sig_rows, hop), jnp.bfloat16)
            for v in range(nv):
                zbuf[bb][:, v * hop:(v + 1) * hop] = ybuf[v:v + zrows, :]
        for bb in range(bpg):
            d1 = jnp.dot(zbuf[bb][0:mp, :], c1_ref[...],
                         preferred_element_type=jnp.float32)
            d2 = jnp.dot(zbuf[bb][0:mp, 0:nh * hop], c2_ref[...],
                         preferred_element_type=jnp.float32)
            lo_a = (d1[aS:aS + n_frames, 0:32]
                    + d1[aS + k1:aS + k1 + n_frames, 32:64]
                    + d1[aS + 2 * k1:aS + 2 * k1 + n_frames, 64:96]
                    + d1[aS + 3 * k1:aS + 3 * k1 + n_frames, 96:128])
            lo_c = (d1[cS:cS + n_frames, 128:160]
                    + d1[cS + k1:cS + k1 + n_frames, 160:192])
            lo_b = d1[bS:bS + n_frames, 192:256]
            hi = d2[hr0:hr0 + n_frames, 0:kout - 128]
            o_ref[bb] = jnp.concatenate([lo_a, lo_c, lo_b, hi], axis=1)

    return pl.pallas_call(
        body,
        out_shape=jax.ShapeDtypeStruct((batch, n_frames, kout), jnp.float32),
        grid=(batch // bpg,),
        in_specs=[
            pl.BlockSpec((bpg, sig_rows, hop), lambda b: (b, 0, 0)),
            pl.BlockSpec(c_d1.shape, lambda b: (0, 0)),
            pl.BlockSpec(c_d2.shape, lambda b: (0, 0)),
        ],
        out_specs=pl.BlockSpec((bpg, n_frames, kout), lambda b: (b, 0, 0)),
        scratch_shapes=[
            pltpu.VMEM((yrows, hop), jnp.bfloat16),
            [pltpu.VMEM((zrows, nv * hop), jnp.bfloat16)
             for _ in range(bpg)],
        ],
        compiler_params=pltpu.CompilerParams(
            dimension_semantics=("parallel",)),
    )(x4, c_d1, c_d2)


def kernel(x):
    cst = _cqt_constants()
    n_bins, fft_len = cst["n_bins"], cst["fft_len"]
    s0 = cst["s0"]

    x = jnp.asarray(x, jnp.float32)
    lead, t_len = x.shape[:-1], x.shape[-1]
    x2 = x.reshape(-1, t_len)
    batch = x2.shape[0]
    n_frames = (t_len - 1) // _HOP + 1

    sig_rows = -(-t_len // _HOP)
    if t_len % _HOP:
        x2 = jnp.pad(x2, ((0, 0), (0, sig_rows * _HOP - t_len)))
    x4 = x2.reshape(batch, sig_rows, _HOP)

    lead_zero = fft_len // 2 // _HOP - s0
    out = _cqt_pallas(x4, cst["c_d1"], cst["c_d2"], n_frames=n_frames,
                      lead_zero=lead_zero, sig_rows=sig_rows, cst=cst,
                      kout=2 * n_bins)
    return out.reshape(*lead, n_frames, n_bins, 2)


# speedup vs baseline: 1.2314x; 1.0128x over previous
import functools
import math

import numpy as np
import jax
import jax.numpy as jnp
from jax.experimental import pallas as pl
from jax.experimental.pallas import tpu as pltpu

_SR = 22050
_F_MIN = 32.7
_BPO = 12
_HOP = 256


@functools.lru_cache(maxsize=None)
def _cqt_constants():
    f_max = _SR / 2.0
    q = 1.0 / (2.0 ** (1.0 / _BPO) - 1.0)
    n_bins = math.ceil(_BPO * math.log2(f_max / _F_MIN))
    fft_len = 1 << (int(math.ceil(q * _SR / _F_MIN)) - 1).bit_length()

    temporal = np.zeros((n_bins, fft_len), dtype=np.complex128)
    for k in range(n_bins):
        f_k = _F_MIN * 2.0 ** (k / _BPO)
        n_k = 2 * round(q * _SR / f_k / 2) + 1
        n = np.arange(-(n_k - 1) // 2, (n_k - 1) // 2 + 1)
        w = np.hamming(n_k) / n_k
        start = fft_len // 2 + n[0]
        temporal[k, start:start + n_k] = w * np.exp(2j * np.pi * q / n_k * n)
    spectral = np.fft.fft(temporal, axis=-1) / fft_len
    folded = np.fft.fft(spectral, axis=-1).T

    c_int = np.zeros((fft_len, 2 * n_bins), dtype=np.float64)
    c_int[:, 0::2] = folded.real
    c_int[:, 1::2] = folded.imag

    row_amp = np.abs(c_int).max(axis=1)
    nz = np.nonzero(row_amp > row_amp.max() * 1e-7)[0]
    s0 = int(nz[0]) // _HOP
    s1 = int(nz[-1]) // _HOP + 1
    ns = s1 - s0

    c_pad = np.zeros((fft_len, 256), np.float64)
    c_pad[:, :2 * n_bins] = c_int
    hi_amp = np.abs(c_pad[:, 128:]).max(axis=1)
    hz = np.nonzero(hi_amp > row_amp.max() * 1e-7)[0]
    hr0 = int(hz[0]) // _HOP - s0
    hr1 = int(hz[-1]) // _HOP + 1 - s0

    def _band(cols):
        amp = np.abs(c_pad[:, cols]).max(axis=1)
        nzc = np.nonzero(amp > row_amp.max() * 1e-7)[0]
        return int(nzc[0]) // _HOP - s0, int(nzc[-1]) // _HOP + 1 - s0

    aS, aE = _band(slice(0, 32))
    cS, cE = _band(slice(32, 64))
    bS, bE = _band(slice(64, 128))
    k1 = max(-(-(aE - aS) // 4), -(-(cE - cS) // 2), bE - bS)
    nv = k1
    c_d1 = np.zeros((k1 * _HOP, 256), np.float32)
    for t in range(k1):
        for m in range(4):
            s = aS + m * k1 + t
            if s < min(aE, ns):
                c_d1[t * _HOP:(t + 1) * _HOP, 32 * m:32 * (m + 1)] = (
                    c_pad[(s0 + s) * _HOP:(s0 + s + 1) * _HOP, :32])
        for m in range(2):
            s = cS + m * k1 + t
            if s < min(cE, ns):
                c_d1[t * _HOP:(t + 1) * _HOP, 128 + 32 * m:160 + 32 * m] = (
                    c_pad[(s0 + s) * _HOP:(s0 + s + 1) * _HOP, 32:64])
        s = bS + t
        if s < bE:
            c_d1[t * _HOP:(t + 1) * _HOP, 192:256] = (
                c_pad[(s0 + s) * _HOP:(s0 + s + 1) * _HOP, 64:128])
    nh = hr1 - hr0
    c_d2 = np.zeros((nh * _HOP, 128), np.float32)
    for t in range(nh):
        c_d2[t * _HOP:(t + 1) * _HOP] = c_pad[(s0 + hr0 + t) * _HOP:
                                              (s0 + hr0 + t + 1) * _HOP, 128:]
    return {
        "n_bins": n_bins,
        "fft_len": fft_len,
        "s0": s0,
        "ns": ns,
        "k1": k1,
        "aS": aS,
        "cS": cS,
        "bS": bS,
        "hr0": hr0,
        "nh": nh,
        "nv": nv,
        "c_d1": jnp.asarray(c_d1, jnp.bfloat16),
        "c_d2": jnp.asarray(c_d2, jnp.bfloat16),
    }


def _cqt_pallas(x4, c_d1, c_d2, *, n_frames, lead_zero, sig_rows, cst, kout):
    batch = x4.shape[0]
    hop = x4.shape[2]
    k1, aS, cS, bS = cst["k1"], cst["aS"], cst["cS"], cst["bS"]
    hr0, nh, nv = cst["hr0"], cst["nh"], cst["nv"]
    off_max = max(aS + 3 * k1, cS + k1, bS, hr0)
    mp = -(-(n_frames + off_max) // 8) * 8
    zrows = -(-mp // 16) * 16
    yrows = -(-(zrows + nv - 1) // 16) * 16

    bpg = next((b for b in (8, 4, 2, 1) if batch % b == 0), 1)

    def body(x_ref, c1_ref, c2_ref, o_ref, ybuf, zbuf):
        for bb in range(bpg):
            ybuf[:lead_zero, :] = jnp.zeros((lead_zero, hop), jnp.bfloat16)
            ybuf[lead_zero:lead_zero + sig_rows, :] = (
                x_ref[bb].astype(jnp.bfloat16))
            ybuf[lead_zero + sig_rows:, :] = jnp.zeros(
                (yrows - lead_zero - sig_rows, hop), jnp.bfloat16)
            for v in range(nv):
                zbuf[bb][:, v * hop:(v + 1) * hop] = ybuf[v:v + zrows, :]
        for bb in range(bpg):
            d1 = jnp.dot(zbuf[bb][0:mp, :], c1_ref[...],
                         preferred_element_type=jnp.float32)
            d2 = jnp.dot(zbuf[bb][0:mp, 0:nh * hop], c2_ref[...],
                         preferred_element_type=jnp.float32)
            lo_a = (d1[aS:aS + n_frames, 0:32]
                    + d1[aS + k1:aS + k1 + n_frames, 32:64]
                    + d1[aS + 2 * k1:aS + 2 * k1 + n_frames, 64:96]
                    + d1[aS + 3 * k1:aS + 3 * k1 + n_frames, 96:128])
            lo_c = (d1[cS:cS + n_frames, 128:160]
                    + d1[cS + k1:cS + k1 + n_frames, 160:192])
            lo_b = d1[bS:bS + n_frames, 192:256]
            hi = d2[hr0:hr0 + n_frames, 0:kout - 128]
            o_ref[bb] = jnp.concatenate([lo_a, lo_c, lo_b, hi], axis=1)

    return pl.pallas_call(
        body,
        out_shape=jax.ShapeDtypeStruct((batch, n_frames, kout), jnp.float32),
        grid=(batch // bpg,),
        in_specs=[
            pl.BlockSpec((bpg, sig_rows, hop), lambda b: (b, 0, 0)),
            pl.BlockSpec(c_d1.shape, lambda b: (0, 0)),
            pl.BlockSpec(c_d2.shape, lambda b: (0, 0)),
        ],
        out_specs=pl.BlockSpec((bpg, n_frames, kout), lambda b: (b, 0, 0)),
        scratch_shapes=[
            pltpu.VMEM((yrows, hop), jnp.bfloat16),
            [pltpu.VMEM((zrows, nv * hop), jnp.bfloat16)
             for _ in range(bpg)],
        ],
        compiler_params=pltpu.CompilerParams(
            dimension_semantics=("parallel",)),
    )(x4, c_d1, c_d2)


def kernel(x):
    cst = _cqt_constants()
    n_bins, fft_len = cst["n_bins"], cst["fft_len"]
    s0 = cst["s0"]

    x = jnp.asarray(x, jnp.float32)
    lead, t_len = x.shape[:-1], x.shape[-1]
    x2 = x.reshape(-1, t_len)
    batch = x2.shape[0]
    n_frames = (t_len - 1) // _HOP + 1

    sig_rows = -(-t_len // _HOP)
    if t_len % _HOP:
        x2 = jnp.pad(x2, ((0, 0), (0, sig_rows * _HOP - t_len)))
    x4 = x2.reshape(batch, sig_rows, _HOP)

    lead_zero = fft_len // 2 // _HOP - s0
    out = _cqt_pallas(x4, cst["c_d1"], cst["c_d2"], n_frames=n_frames,
                      lead_zero=lead_zero, sig_rows=sig_rows, cst=cst,
                      kout=2 * n_bins)
    return out.reshape(*lead, n_frames, n_bins, 2)
